# baseline (speedup 1.0000x reference)
import functools

import jax
import jax.numpy as jnp
from jax.experimental import pallas as pl
from jax.experimental.pallas import tpu as pltpu

N1, N2, N3 = 8928, 2232, 558
K = 6
B = 32


def _matmul_kernel(x_ref, w_ref, b_ref, o_ref, *, relu):
    acc = jnp.dot(x_ref[...], w_ref[...], preferred_element_type=jnp.float32)
    acc = acc + b_ref[...]
    if relu:
        acc = jnp.maximum(acc, 0.0)
    o_ref[...] = acc


def _dense(x, w, b, relu=False):
    m, k = x.shape
    k2, n = w.shape
    bm = min(m, 2048)
    bn = min(n, 1024)
    grid = (pl.cdiv(m, bm), pl.cdiv(n, bn))
    return pl.pallas_call(
        functools.partial(_matmul_kernel, relu=relu),
        grid=grid,
        in_specs=[
            pl.BlockSpec((bm, k), lambda i, j: (i, 0)),
            pl.BlockSpec((k, bn), lambda i, j: (0, j)),
            pl.BlockSpec((bn,), lambda i, j: (j,)),
        ],
        out_specs=pl.BlockSpec((bm, bn), lambda i, j: (i, j)),
        out_shape=jax.ShapeDtypeStruct((m, n), jnp.float32),
    )(x, w, b)


def _spmm(rows, cols, vals, n_rows, x):
    g = x[:, cols, :] * vals[None, :, None]
    y = jax.ops.segment_sum(jnp.transpose(g, (1, 0, 2)), rows, num_segments=n_rows)
    return jnp.transpose(y, (1, 0, 2))


def _cheb_conv(x, rows, cols, vals, n, W, b):
    xs = [x]
    x1 = _spmm(rows, cols, vals, n, x)
    xs.append(x1)
    for _ in range(2, K):
        xs.append(2.0 * _spmm(rows, cols, vals, n, xs[-1]) - xs[-2])
    xk = jnp.concatenate(xs, axis=-1)
    Bn, n_, kf = xk.shape
    out = _dense(xk.reshape(Bn * n_, kf), W, b)
    return out.reshape(Bn, n_, W.shape[1])


def _cheb_res_block(x, rows, cols, vals, n, c1W, c1b, c2W, c2b, scW, scb):
    h = jax.nn.relu(_cheb_conv(x, rows, cols, vals, n, c1W, c1b))
    h = _cheb_conv(h, rows, cols, vals, n, c2W, c2b)
    Bn, n_, fi = x.shape
    sc = _dense(x.reshape(Bn * n_, fi), scW, scb).reshape(Bn, n_, scW.shape[1])
    return jax.nn.relu(h + sc)


def kernel(mesh, lap1_rows, lap1_cols, lap1_vals, lap2_rows, lap2_cols, lap2_vals, up_rows, up_cols, up_vals, dn1_rows, dn1_cols, dn1_vals, dn2_rows, dn2_cols, dn2_vals, fc_W, fc_b, b1_c1W, b1_c1b, b1_c2W, b1_c2b, b1_scW, b1_scb, b2_c1W, b2_c1b, b2_c2W, b2_c2b, b2_scW, b2_scb, b3_c1W, b3_c1b, b3_c2W, b3_c2b, b3_scW, b3_scb, last_W, last_b):
    Bn = mesh.shape[0]
    layer1 = _dense(mesh, fc_W, fc_b, relu=True).reshape(Bn, N2, 3)
    layer2 = _cheb_res_block(layer1, lap2_rows, lap2_cols, lap2_vals, N2, b1_c1W, b1_c1b, b1_c2W, b1_c2b, b1_scW, b1_scb)
    layer3 = _spmm(up_rows, up_cols, up_vals, N1, layer2)
    layer3 = _cheb_res_block(layer3, lap1_rows, lap1_cols, lap1_vals, N1, b2_c1W, b2_c1b, b2_c2W, b2_c2b, b2_scW, b2_scb)
    layer4 = _spmm(dn1_rows, dn1_cols, dn1_vals, N2, layer3)
    layer4 = _cheb_res_block(layer4, lap2_rows, lap2_cols, lap2_vals, N2, b3_c1W, b3_c1b, b3_c2W, b3_c2b, b3_scW, b3_scb)
    layer5 = _spmm(dn2_rows, dn2_cols, dn2_vals, N3, layer4)
    layer5 = layer5.reshape(Bn, N3 * 3)
    out = _dense(layer5, last_W, last_b)
    return out.reshape(Bn, 64, 1)


# speedup vs baseline: 11.5073x; 11.5073x over previous
import functools

import jax
import jax.numpy as jnp
from jax import lax
from jax.experimental import pallas as pl
from jax.experimental.pallas import tpu as pltpu
from jax.experimental.pallas import tpu_sc as plsc

N1, N2, N3 = 8928, 2232, 558
K = 6
B = 32

N1P, N2P, N3P = 8960, 2304, 640
C = 128
WCH = 64


def _make_spmm_sc(n_iter, nip, nop, q_tot, dq, nch):
    cpt = nch // 16
    rpt = nop // 16
    nsl = dq // 16
    mesh = plsc.VectorSubcoreMesh(core_axis_name="c", subcore_axis_name="s",
                                  num_cores=2, num_subcores=16)

    @functools.partial(
        pl.kernel,
        out_type=jax.ShapeDtypeStruct((n_iter * q_tot * nop, dq), jnp.float32),
        mesh=mesh,
        scratch_types=[
            pltpu.VMEM_SHARED((nop, dq), jnp.float32),
            pltpu.VMEM((3, C), jnp.int32),
            pltpu.VMEM((C,), jnp.int32),
            pltpu.VMEM((C, dq), jnp.float32),
            pltpu.VMEM((WCH, dq), jnp.float32),
            pltpu.VMEM((WCH, dq), jnp.float32),
            pltpu.VMEM((WCH, dq), jnp.float32),
            pltpu.SemaphoreType.DMA,
        ],
        compiler_params=pltpu.CompilerParams(needs_layout_passes=False,
                                             use_tc_tiling_on_sc=False),
    )
    def spmm(x0_hbm, pack_hbm, out_hbm, y_sp, pbuf, colv, gbuf, wbuf, xpbuf,
             zbuf, sem):
        c = lax.axis_index("c")
        s = lax.axis_index("s")

        def zrow(i, _):
            for d in range(nsl):
                zbuf[i, pl.ds(d * 16, 16)] = jnp.zeros((16,), jnp.float32)
            return 0
        lax.fori_loop(0, WCH, zrow, 0)

        def scatter_chunks(src_hbm, gbase):
            def chunk_body(ch, _):
                cid = s * cpt + ch
                pltpu.sync_copy(pack_hbm.at[cid], pbuf)
                for i in range(C // 16):
                    colv[pl.ds(i * 16, 16)] = (
                        pbuf[1, pl.ds(i * 16, 16)] + gbase)
                pltpu.async_copy(src_hbm.at[colv], gbuf, sem).wait()

                def e_body(e, _):
                    vb = plsc.load_gather(
                        pbuf.at[2], [jnp.full((16,), e, jnp.int32)])
                    v = plsc.bitcast(vb, jnp.float32)
                    for d in range(nsl):
                        gbuf[e, pl.ds(d * 16, 16)] = (
                            gbuf[e, pl.ds(d * 16, 16)] * v)
                    return 0
                lax.fori_loop(0, C, e_body, 0)
                pltpu.sync_copy(gbuf, y_sp.at[pbuf.at[0]], add=True)
                return 0
            lax.fori_loop(0, cpt, chunk_body, 0)

        for qi in range(q_tot // 2):
            q = 2 * qi + c
            for k in range(1, n_iter + 1):
                for w in range(0, rpt, WCH):
                    cnt = min(WCH, rpt - w)
                    pltpu.sync_copy(
                        zbuf.at[pl.ds(0, cnt)],
                        y_sp.at[pl.ds(s * rpt + w, cnt)])
                plsc.subcore_barrier()

                if k == 1:
                    scatter_chunks(x0_hbm, q * nip)
                else:
                    scatter_chunks(out_hbm, ((k - 2) * q_tot + q) * nop)
                plsc.subcore_barrier()

                out_base = ((k - 1) * q_tot + q) * nop + s * rpt
                if k == 1:
                    pltpu.sync_copy(
                        y_sp.at[pl.ds(s * rpt, rpt)],
                        out_hbm.at[pl.ds(out_base, rpt)])
                else:
                    if k == 2:
                        pv_hbm, pv_base = x0_hbm, q * nip + s * rpt
                    else:
                        pv_hbm = out_hbm
                        pv_base = ((k - 3) * q_tot + q) * nop + s * rpt
                    for w in range(0, rpt, WCH):
                        cnt = min(WCH, rpt - w)
                        pltpu.sync_copy(
                            y_sp.at[pl.ds(s * rpt + w, cnt)],
                            wbuf.at[pl.ds(0, cnt)])
                        pltpu.sync_copy(
                            pv_hbm.at[pl.ds(pv_base + w, cnt)],
                            xpbuf.at[pl.ds(0, cnt)])

                        def axpy_row(i, _):
                            for d in range(nsl):
                                sl = pl.ds(d * 16, 16)
                                wbuf[i, sl] = (2.0 * wbuf[i, sl]
                                               - xpbuf[i, sl])
                            return 0
                        lax.fori_loop(0, cnt, axpy_row, 0)
                        pltpu.sync_copy(
                            wbuf.at[pl.ds(0, cnt)],
                            out_hbm.at[pl.ds(out_base + w, cnt)])
                plsc.subcore_barrier()

    return spmm


def _pack_coo(rows, cols, vals):
    nnz = rows.shape[0]
    nch = 16 * ((nnz + 16 * C - 1) // (16 * C))
    pad = nch * C - nnz
    r = jnp.pad(rows, (0, pad))
    co = jnp.pad(cols, (0, pad))
    v = jnp.pad(vals, (0, pad))
    vb = lax.bitcast_convert_type(v, jnp.int32)
    pk = jnp.stack([r, co, vb], axis=1).reshape(nch, C, 3)
    return jnp.transpose(pk, (0, 2, 1)), nch


def _matmul_kernel(x_ref, w_ref, b_ref, o_ref, *, relu):
    acc = jnp.dot(x_ref[...], w_ref[...], preferred_element_type=jnp.float32)
    acc = acc + b_ref[...]
    if relu:
        acc = jnp.maximum(acc, 0.0)
    o_ref[...] = acc


def _dense(x, w, b, relu=False):
    m, k = x.shape
    _, n = w.shape
    bm = min(m, 2048)
    bn = min(n, 1024)
    grid = (pl.cdiv(m, bm), pl.cdiv(n, bn))
    return pl.pallas_call(
        functools.partial(_matmul_kernel, relu=relu),
        grid=grid,
        in_specs=[
            pl.BlockSpec((bm, k), lambda i, j: (i, 0)),
            pl.BlockSpec((k, bn), lambda i, j: (0, j)),
            pl.BlockSpec((bn,), lambda i, j: (j,)),
        ],
        out_specs=pl.BlockSpec((bm, bn), lambda i, j: (i, j)),
        out_shape=jax.ShapeDtypeStruct((m, n), jnp.float32),
    )(x, w, b)


def _cheb_mm_kernel(x_ref, xs_ref, w_ref, b_ref, o_ref, *, f, relu):
    acc = jnp.dot(x_ref[...], w_ref[pl.ds(0, f), :],
                  preferred_element_type=jnp.float32)
    for j in range(K - 1):
        acc = acc + jnp.dot(xs_ref[j], w_ref[pl.ds((j + 1) * f, f), :],
                            preferred_element_type=jnp.float32)
    acc = acc + b_ref[...]
    if relu:
        acc = jnp.maximum(acc, 0.0)
    o_ref[...] = acc


def _cheb_mm_sc_kernel(x_ref, xs_ref, w_ref, b_ref, xsc_ref, scw_ref,
                       scb_ref, o_ref, *, f, relu):
    acc = jnp.dot(x_ref[...], w_ref[pl.ds(0, f), :],
                  preferred_element_type=jnp.float32)
    for j in range(K - 1):
        acc = acc + jnp.dot(xs_ref[j], w_ref[pl.ds((j + 1) * f, f), :],
                            preferred_element_type=jnp.float32)
    acc = acc + b_ref[...]
    acc = acc + jnp.dot(xsc_ref[...], scw_ref[...],
                        preferred_element_type=jnp.float32) + scb_ref[...]
    if relu:
        acc = jnp.maximum(acc, 0.0)
    o_ref[...] = acc


def _cheb_mm(x, xs, w, b, xsc=None, scw=None, scb=None, relu=False):
    m, f = x.shape
    fo = w.shape[1]
    bm = 2048
    grid = (pl.cdiv(m, bm),)
    in_specs = [
        pl.BlockSpec((bm, f), lambda i: (i, 0)),
        pl.BlockSpec((K - 1, bm, f), lambda i: (0, i, 0)),
        pl.BlockSpec((K * f, fo), lambda i: (0, 0)),
        pl.BlockSpec((fo,), lambda i: (0,)),
    ]
    args = [x, xs, w, b]
    if xsc is None:
        kern = functools.partial(_cheb_mm_kernel, f=f, relu=relu)
    else:
        fi = xsc.shape[1]
        in_specs += [
            pl.BlockSpec((bm, fi), lambda i: (i, 0)),
            pl.BlockSpec((fi, fo), lambda i: (0, 0)),
            pl.BlockSpec((fo,), lambda i: (0,)),
        ]
        args += [xsc, scw, scb]
        kern = functools.partial(_cheb_mm_sc_kernel, f=f, relu=relu)
    return pl.pallas_call(
        kern,
        grid=grid,
        in_specs=in_specs,
        out_specs=pl.BlockSpec((bm, fo), lambda i: (i, 0)),
        out_shape=jax.ShapeDtypeStruct((m, fo), jnp.float32),
    )(*args)


def _to_q(x, q):
    b, n, f = x.shape
    return jnp.transpose(x.reshape(q, b // q, n, f), (0, 2, 1, 3))


def _q2_to_q4(x):
    q, n, bq, f = x.shape
    return jnp.transpose(x.reshape(2, n, 2, 8, f),
                         (0, 2, 1, 3, 4)).reshape(4, n, 8, f)


def _q4_to_q2(x):
    q, n, bq, f = x.shape
    return jnp.transpose(x.reshape(2, 2, n, 8, f),
                         (0, 2, 1, 3, 4)).reshape(2, n, 16, f)


def _pad_n(x, npad):
    return jnp.pad(x, ((0, 0), (0, npad - x.shape[1]), (0, 0), (0, 0)))


def kernel(mesh, lap1_rows, lap1_cols, lap1_vals, lap2_rows, lap2_cols, lap2_vals, up_rows, up_cols, up_vals, dn1_rows, dn1_cols, dn1_vals, dn2_rows, dn2_cols, dn2_vals, fc_W, fc_b, b1_c1W, b1_c1b, b1_c2W, b1_c2b, b1_scW, b1_scb, b2_c1W, b2_c1b, b2_c2W, b2_c2b, b2_scW, b2_scb, b3_c1W, b3_c1b, b3_c2W, b3_c2b, b3_scW, b3_scb, last_W, last_b):
    pk1, nch1 = _pack_coo(lap1_rows, lap1_cols, lap1_vals)
    pk2, nch2 = _pack_coo(lap2_rows, lap2_cols, lap2_vals)
    pku, nchu = _pack_coo(up_rows, up_cols, up_vals)
    pkd1, nchd1 = _pack_coo(dn1_rows, dn1_cols, dn1_vals)
    pkd2, nchd2 = _pack_coo(dn2_rows, dn2_cols, dn2_vals)

    m2 = 2 * N2P * 16
    m1 = 4 * N1P * 8

    l1 = _dense(mesh, fc_W, fc_b, relu=True).reshape(B, N2, 3)
    l1q2 = _pad_n(_to_q(l1, 2), N2P)

    spmm2_48 = _make_spmm_sc(5, N2P, N2P, 2, 48, nch2)
    xs = spmm2_48(l1q2.reshape(2 * N2P, 48), pk2)
    h = _cheb_mm(l1q2.reshape(m2, 3), xs.reshape(5, m2, 3),
                 b1_c1W, b1_c1b, relu=True)
    hq4 = _q2_to_q4(h.reshape(2, N2P, 16, 16))
    spmm2_128 = _make_spmm_sc(5, N2P, N2P, 4, 128, nch2)
    xs = spmm2_128(hq4.reshape(4 * N2P, 128), pk2)
    l2 = _cheb_mm(hq4.reshape(m2, 16), xs.reshape(5, m2, 16),
                  b1_c2W, b1_c2b,
                  xsc=_q2_to_q4(l1q2).reshape(m2, 3),
                  scw=b1_scW, scb=b1_scb, relu=True)
    l2q4 = l2.reshape(4, N2P, 8, 16)

    spmm_up = _make_spmm_sc(1, N2P, N1P, 4, 128, nchu)
    l3q4 = spmm_up(l2q4.reshape(4 * N2P, 128), pku).reshape(4, N1P, 8, 16)

    spmm1_128 = _make_spmm_sc(5, N1P, N1P, 4, 128, nch1)
    xs = spmm1_128(l3q4.reshape(4 * N1P, 128), pk1)
    h = _cheb_mm(l3q4.reshape(m1, 16), xs.reshape(5, m1, 16),
                 b2_c1W, b2_c1b, relu=True)
    hq4 = h.reshape(4, N1P, 8, 16)
    xs = spmm1_128(hq4.reshape(4 * N1P, 128), pk1)
    l3 = _cheb_mm(h, xs.reshape(5, m1, 16), b2_c2W, b2_c2b,
                  xsc=l3q4.reshape(m1, 16), scw=b2_scW, scb=b2_scb,
                  relu=True)
    l3q4 = l3.reshape(4, N1P, 8, 16)

    spmm_dn1 = _make_spmm_sc(1, N1P, N2P, 4, 128, nchd1)
    l4q4 = spmm_dn1(l3q4.reshape(4 * N1P, 128), pkd1).reshape(4, N2P, 8, 16)

    xs = spmm2_128(l4q4.reshape(4 * N2P, 128), pk2)
    h = _cheb_mm(l4q4.reshape(m2, 16), xs.reshape(5, m2, 16),
                 b3_c1W, b3_c1b, relu=True)
    hq2 = _q4_to_q2(h.reshape(4, N2P, 8, 3))
    xs = spmm2_48(hq2.reshape(2 * N2P, 48), pk2)
    l4 = _cheb_mm(hq2.reshape(m2, 3), xs.reshape(5, m2, 3),
                  b3_c2W, b3_c2b,
                  xsc=_q4_to_q2(l4q4).reshape(m2, 16),
                  scw=b3_scW, scb=b3_scb, relu=True)
    l4q2 = l4.reshape(2, N2P, 16, 3)

    spmm_dn2 = _make_spmm_sc(1, N2P, N3P, 2, 48, nchd2)
    l5 = spmm_dn2(l4q2.reshape(2 * N2P, 48), pkd2).reshape(2, N3P, 16, 3)

    l5 = jnp.transpose(l5[:, :N3], (0, 2, 1, 3)).reshape(B, N3 * 3)
    out = _dense(l5, last_W, last_b)
    return out.reshape(B, 64, 1)


# speedup vs baseline: 13.1683x; 1.1444x over previous
import functools

import jax
import jax.numpy as jnp
from jax import lax
from jax.experimental import pallas as pl
from jax.experimental.pallas import tpu as pltpu
from jax.experimental.pallas import tpu_sc as plsc

N1, N2, N3 = 8928, 2232, 558
K = 6
B = 32

N1P, N2P, N3P = 8960, 2304, 640
C = 128
WCH = 64


def _make_spmm_sc(n_iter, nip, nop, q_tot, dq, nch):
    rpt = nop // 16
    nsl = dq // 16
    mesh = plsc.VectorSubcoreMesh(core_axis_name="c", subcore_axis_name="s",
                                  num_cores=2, num_subcores=16)

    @functools.partial(
        pl.kernel,
        out_type=jax.ShapeDtypeStruct((n_iter * q_tot * nop, dq), jnp.float32),
        mesh=mesh,
        scratch_types=[
            pltpu.VMEM((rpt, dq), jnp.float32),
            pltpu.VMEM((3, C), jnp.int32),
            pltpu.VMEM((3, C), jnp.int32),
            pltpu.VMEM((C,), jnp.int32),
            pltpu.VMEM((C,), jnp.int32),
            pltpu.VMEM((C, dq), jnp.float32),
            pltpu.VMEM((C, dq), jnp.float32),
            pltpu.VMEM((WCH, dq), jnp.float32),
            pltpu.VMEM((2, 16), jnp.int32),
            pltpu.SemaphoreType.DMA,
            pltpu.SemaphoreType.DMA,
            pltpu.SemaphoreType.DMA,
            pltpu.SemaphoreType.DMA,
        ],
        compiler_params=pltpu.CompilerParams(needs_layout_passes=False,
                                             use_tc_tiling_on_sc=False),
    )
    def spmm(x0_hbm, pack_hbm, meta_hbm, out_hbm, slab, pb0, pb1, cv0, cv1,
             gb0, gb1, xpbuf, metav, semp0, semp1, semg0, semg1):
        c = lax.axis_index("c")
        s = lax.axis_index("s")
        pbufs, colvs, gbufs = (pb0, pb1), (cv0, cv1), (gb0, gb1)
        semps, semgs = (semp0, semp1), (semg0, semg1)

        pltpu.sync_copy(meta_hbm, metav)
        lane = lax.iota(jnp.int32, 16)
        is_me = lane == s
        zero16 = jnp.zeros((16,), jnp.int32)
        c0 = jnp.max(jnp.where(is_me, metav[0, :], zero16))
        trip = jnp.max(jnp.where(is_me, metav[1, :], zero16))
        base_row = s * rpt

        def zero_slab():
            def zrow(i, _):
                for d in range(nsl):
                    slab[i, pl.ds(d * 16, 16)] = jnp.zeros((16,), jnp.float32)
                return 0
            lax.fori_loop(0, rpt, zrow, 0)

        def process(b):
            pbuf, gbuf = pbufs[b], gbufs[b]

            def e_body(e, _):
                eidx = jnp.full((16,), e, jnp.int32)
                rv = plsc.load_gather(pbuf.at[0], [eidx])
                vv = plsc.bitcast(plsc.load_gather(pbuf.at[2], [eidx]),
                                  jnp.float32)
                rloc = rv - base_row
                mask = (rloc >= 0) & (rloc < rpt)
                for d in range(nsl):
                    g = gbuf[e, pl.ds(d * 16, 16)] * vv
                    plsc.addupdate_scatter(
                        slab, [rloc, lane + d * 16], g, mask=mask)
                return 0
            lax.fori_loop(0, C, e_body, 0)

        def chunk_loop(src_hbm, gbase):
            @pl.when(trip > 0)
            def _():
                pltpu.make_async_copy(pack_hbm.at[c0], pb0, semp0).start()

            def half(ch, b, bo):
                pbuf, colv = pbufs[b], colvs[b]

                @pl.when(ch < trip)
                def _():
                    pltpu.make_async_copy(
                        pack_hbm.at[c0 + ch], pbuf, semps[b]).wait()
                    for i in range(C // 16):
                        colv[pl.ds(i * 16, 16)] = (
                            pbuf[1, pl.ds(i * 16, 16)] + gbase)
                    pltpu.make_async_copy(
                        src_hbm.at[colv], gbufs[b], semgs[b]).start()

                @pl.when((ch >= 1) & (ch <= trip))
                def _():
                    pltpu.make_async_copy(
                        src_hbm.at[colvs[bo]], gbufs[bo], semgs[bo]).wait()
                    process(bo)

                @pl.when(ch + 1 < trip)
                def _():
                    pltpu.make_async_copy(
                        pack_hbm.at[c0 + ch + 1], pbufs[bo], semps[bo]
                    ).start()

            def pair(p, _):
                half(2 * p, 0, 1)
                half(2 * p + 1, 1, 0)
                return 0
            lax.fori_loop(0, (trip + 2) // 2, pair, 0)

        for qi in range(q_tot // 2):
            q = 2 * qi + c
            for k in range(1, n_iter + 1):
                zero_slab()
                if k == 1:
                    chunk_loop(x0_hbm, q * nip)
                else:
                    chunk_loop(out_hbm, ((k - 2) * q_tot + q) * nop)

                out_base = ((k - 1) * q_tot + q) * nop + s * rpt
                if k > 1:
                    if k == 2:
                        pv_hbm, pv_base = x0_hbm, q * nip + s * rpt
                    else:
                        pv_hbm = out_hbm
                        pv_base = ((k - 3) * q_tot + q) * nop + s * rpt
                    for w in range(0, rpt, WCH):
                        cnt = min(WCH, rpt - w)
                        pltpu.sync_copy(
                            pv_hbm.at[pl.ds(pv_base + w, cnt)],
                            xpbuf.at[pl.ds(0, cnt)])

                        def axpy_row(i, _):
                            for d in range(nsl):
                                sl = pl.ds(d * 16, 16)
                                slab[w + i, sl] = (2.0 * slab[w + i, sl]
                                                   - xpbuf[i, sl])
                            return 0
                        lax.fori_loop(0, cnt, axpy_row, 0)
                pltpu.sync_copy(slab, out_hbm.at[pl.ds(out_base, rpt)])
                plsc.subcore_barrier()

    return spmm


def _pack_coo(rows, cols, vals, nop):
    nnz = rows.shape[0]
    r, co, v = lax.sort((rows, cols, vals), num_keys=1)
    nch = (nnz + C - 1) // C
    pad = nch * C - nnz
    r = jnp.pad(r, (0, pad), constant_values=nop)
    co = jnp.pad(co, (0, pad))
    v = jnp.pad(v, (0, pad))
    vb = lax.bitcast_convert_type(v, jnp.int32)
    pk = jnp.transpose(jnp.stack([r, co, vb], axis=1).reshape(nch, C, 3),
                       (0, 2, 1))
    rpt = nop // 16
    bounds = jnp.arange(17, dtype=jnp.int32) * rpt
    off = jnp.searchsorted(r, bounds).astype(jnp.int32)
    start_ch = off[:16] // C
    end_ch = (off[1:] + C - 1) // C
    cnt = jnp.maximum(end_ch - start_ch, 0)
    cnt = jnp.where(off[1:] > off[:16], cnt, 0)
    meta = jnp.stack([start_ch, cnt]).astype(jnp.int32)
    return pk, meta, nch


def _matmul_kernel(x_ref, w_ref, b_ref, o_ref, *, relu):
    acc = jnp.dot(x_ref[...], w_ref[...], preferred_element_type=jnp.float32)
    acc = acc + b_ref[...]
    if relu:
        acc = jnp.maximum(acc, 0.0)
    o_ref[...] = acc


def _dense(x, w, b, relu=False):
    m, k = x.shape
    _, n = w.shape
    bm = min(m, 2048)
    bn = min(n, 1024)
    grid = (pl.cdiv(m, bm), pl.cdiv(n, bn))
    return pl.pallas_call(
        functools.partial(_matmul_kernel, relu=relu),
        grid=grid,
        in_specs=[
            pl.BlockSpec((bm, k), lambda i, j: (i, 0)),
            pl.BlockSpec((k, bn), lambda i, j: (0, j)),
            pl.BlockSpec((bn,), lambda i, j: (j,)),
        ],
        out_specs=pl.BlockSpec((bm, bn), lambda i, j: (i, j)),
        out_shape=jax.ShapeDtypeStruct((m, n), jnp.float32),
    )(x, w, b)


def _cheb_mm_kernel(x_ref, xs_ref, w_ref, b_ref, o_ref, *, f, relu):
    acc = jnp.dot(x_ref[...], w_ref[pl.ds(0, f), :],
                  preferred_element_type=jnp.float32)
    for j in range(K - 1):
        acc = acc + jnp.dot(xs_ref[j], w_ref[pl.ds((j + 1) * f, f), :],
                            preferred_element_type=jnp.float32)
    acc = acc + b_ref[...]
    if relu:
        acc = jnp.maximum(acc, 0.0)
    o_ref[...] = acc


def _cheb_mm_sc_kernel(x_ref, xs_ref, w_ref, b_ref, xsc_ref, scw_ref,
                       scb_ref, o_ref, *, f, relu):
    acc = jnp.dot(x_ref[...], w_ref[pl.ds(0, f), :],
                  preferred_element_type=jnp.float32)
    for j in range(K - 1):
        acc = acc + jnp.dot(xs_ref[j], w_ref[pl.ds((j + 1) * f, f), :],
                            preferred_element_type=jnp.float32)
    acc = acc + b_ref[...]
    acc = acc + jnp.dot(xsc_ref[...], scw_ref[...],
                        preferred_element_type=jnp.float32) + scb_ref[...]
    if relu:
        acc = jnp.maximum(acc, 0.0)
    o_ref[...] = acc


def _cheb_mm(x, xs, w, b, xsc=None, scw=None, scb=None, relu=False):
    m, f = x.shape
    fo = w.shape[1]
    bm = 2048
    grid = (pl.cdiv(m, bm),)
    in_specs = [
        pl.BlockSpec((bm, f), lambda i: (i, 0)),
        pl.BlockSpec((K - 1, bm, f), lambda i: (0, i, 0)),
        pl.BlockSpec((K * f, fo), lambda i: (0, 0)),
        pl.BlockSpec((fo,), lambda i: (0,)),
    ]
    args = [x, xs, w, b]
    if xsc is None:
        kern = functools.partial(_cheb_mm_kernel, f=f, relu=relu)
    else:
        fi = xsc.shape[1]
        in_specs += [
            pl.BlockSpec((bm, fi), lambda i: (i, 0)),
            pl.BlockSpec((fi, fo), lambda i: (0, 0)),
            pl.BlockSpec((fo,), lambda i: (0,)),
        ]
        args += [xsc, scw, scb]
        kern = functools.partial(_cheb_mm_sc_kernel, f=f, relu=relu)
    return pl.pallas_call(
        kern,
        grid=grid,
        in_specs=in_specs,
        out_specs=pl.BlockSpec((bm, fo), lambda i: (i, 0)),
        out_shape=jax.ShapeDtypeStruct((m, fo), jnp.float32),
    )(*args)


def _to_q(x, q):
    b, n, f = x.shape
    return jnp.transpose(x.reshape(q, b // q, n, f), (0, 2, 1, 3))


def _q2_to_q4(x):
    q, n, bq, f = x.shape
    return jnp.transpose(x.reshape(2, n, 2, 8, f),
                         (0, 2, 1, 3, 4)).reshape(4, n, 8, f)


def _q4_to_q2(x):
    q, n, bq, f = x.shape
    return jnp.transpose(x.reshape(2, 2, n, 8, f),
                         (0, 2, 1, 3, 4)).reshape(2, n, 16, f)


def _pad_n(x, npad):
    return jnp.pad(x, ((0, 0), (0, npad - x.shape[1]), (0, 0), (0, 0)))


def kernel(mesh, lap1_rows, lap1_cols, lap1_vals, lap2_rows, lap2_cols, lap2_vals, up_rows, up_cols, up_vals, dn1_rows, dn1_cols, dn1_vals, dn2_rows, dn2_cols, dn2_vals, fc_W, fc_b, b1_c1W, b1_c1b, b1_c2W, b1_c2b, b1_scW, b1_scb, b2_c1W, b2_c1b, b2_c2W, b2_c2b, b2_scW, b2_scb, b3_c1W, b3_c1b, b3_c2W, b3_c2b, b3_scW, b3_scb, last_W, last_b):
    pk1, mt1, nch1 = _pack_coo(lap1_rows, lap1_cols, lap1_vals, N1P)
    pk2, mt2, nch2 = _pack_coo(lap2_rows, lap2_cols, lap2_vals, N2P)
    pku, mtu, nchu = _pack_coo(up_rows, up_cols, up_vals, N1P)
    pkd1, mtd1, nchd1 = _pack_coo(dn1_rows, dn1_cols, dn1_vals, N2P)
    pkd2, mtd2, nchd2 = _pack_coo(dn2_rows, dn2_cols, dn2_vals, N3P)

    m2 = 2 * N2P * 16
    m1 = 4 * N1P * 8

    l1 = _dense(mesh, fc_W, fc_b, relu=True).reshape(B, N2, 3)
    l1q2 = _pad_n(_to_q(l1, 2), N2P)

    spmm2_48 = _make_spmm_sc(5, N2P, N2P, 2, 48, nch2)
    xs = spmm2_48(l1q2.reshape(2 * N2P, 48), pk2, mt2)
    h = _cheb_mm(l1q2.reshape(m2, 3), xs.reshape(5, m2, 3),
                 b1_c1W, b1_c1b, relu=True)
    hq4 = _q2_to_q4(h.reshape(2, N2P, 16, 16))
    spmm2_128 = _make_spmm_sc(5, N2P, N2P, 4, 128, nch2)
    xs = spmm2_128(hq4.reshape(4 * N2P, 128), pk2, mt2)
    l2 = _cheb_mm(hq4.reshape(m2, 16), xs.reshape(5, m2, 16),
                  b1_c2W, b1_c2b,
                  xsc=_q2_to_q4(l1q2).reshape(m2, 3),
                  scw=b1_scW, scb=b1_scb, relu=True)
    l2q4 = l2.reshape(4, N2P, 8, 16)

    spmm_up = _make_spmm_sc(1, N2P, N1P, 4, 128, nchu)
    l3q4 = spmm_up(l2q4.reshape(4 * N2P, 128), pku, mtu).reshape(4, N1P, 8, 16)

    spmm1_128 = _make_spmm_sc(5, N1P, N1P, 4, 128, nch1)
    xs = spmm1_128(l3q4.reshape(4 * N1P, 128), pk1, mt1)
    h = _cheb_mm(l3q4.reshape(m1, 16), xs.reshape(5, m1, 16),
                 b2_c1W, b2_c1b, relu=True)
    hq4 = h.reshape(4, N1P, 8, 16)
    xs = spmm1_128(hq4.reshape(4 * N1P, 128), pk1, mt1)
    l3 = _cheb_mm(h, xs.reshape(5, m1, 16), b2_c2W, b2_c2b,
                  xsc=l3q4.reshape(m1, 16), scw=b2_scW, scb=b2_scb,
                  relu=True)
    l3q4 = l3.reshape(4, N1P, 8, 16)

    spmm_dn1 = _make_spmm_sc(1, N1P, N2P, 4, 128, nchd1)
    l4q4 = spmm_dn1(l3q4.reshape(4 * N1P, 128), pkd1, mtd1).reshape(4, N2P, 8, 16)

    xs = spmm2_128(l4q4.reshape(4 * N2P, 128), pk2, mt2)
    h = _cheb_mm(l4q4.reshape(m2, 16), xs.reshape(5, m2, 16),
                 b3_c1W, b3_c1b, relu=True)
    hq2 = _q4_to_q2(h.reshape(4, N2P, 8, 3))
    xs = spmm2_48(hq2.reshape(2 * N2P, 48), pk2, mt2)
    l4 = _cheb_mm(hq2.reshape(m2, 3), xs.reshape(5, m2, 3),
                  b3_c2W, b3_c2b,
                  xsc=_q4_to_q2(l4q4).reshape(m2, 16),
                  scw=b3_scW, scb=b3_scb, relu=True)
    l4q2 = l4.reshape(2, N2P, 16, 3)

    spmm_dn2 = _make_spmm_sc(1, N2P, N3P, 2, 48, nchd2)
    l5 = spmm_dn2(l4q2.reshape(2 * N2P, 48), pkd2, mtd2).reshape(2, N3P, 16, 3)

    l5 = jnp.transpose(l5[:, :N3], (0, 2, 1, 3)).reshape(B, N3 * 3)
    out = _dense(l5, last_W, last_b)
    return out.reshape(B, 64, 1)
